# baseline (speedup 1.0000x reference)
import functools

import jax
import jax.numpy as jnp
from jax import lax
from jax.experimental import pallas as pl
from jax.experimental.pallas import tpu as pltpu
from jax.experimental.pallas import tpu_sc as plsc

E = 160000
V = 10000
D = 256
DW = D // 2
NC = 2
NS = 16
NW = NC * NS
EP = 163840
EW = EP // NW
C = 64
C2 = 2 * C
NCHUNK = EW // C
NBUF = 4

_mesh = plsc.VectorSubcoreMesh(core_axis_name="c", subcore_axis_name="s")


@functools.partial(
    pl.kernel,
    mesh=_mesh,
    compiler_params=pltpu.CompilerParams(use_tc_tiling_on_sc=False,
                                         needs_layout_passes=False),
    out_type=jax.ShapeDtypeStruct((NW, NCHUNK, C), jnp.float32),
    scratch_types=[
        pltpu.VMEM((NCHUNK, C2), jnp.int32),
        pltpu.VMEM((NBUF, C2, DW), jnp.float32),
        pltpu.VMEM((NCHUNK, C), jnp.float32),
        pltpu.SemaphoreType.DMA,
        pltpu.SemaphoreType.DMA,
        pltpu.SemaphoreType.DMA,
        pltpu.SemaphoreType.DMA,
    ],
)
def _dot_edges(tb, idx_hbm, out_hbm, idx_v, rows_b, out_v, sem0, sem1, sem2, sem3):
    wid = lax.axis_index("s") * NC + lax.axis_index("c")
    sems = (sem0, sem1, sem2, sem3)

    pltpu.sync_copy(idx_hbm.at[wid], idx_v)

    def fire(g, b):
        pltpu.async_copy(tb.at[idx_v.at[g]], rows_b.at[b], sems[b])

    def wait(b):
        pltpu.make_async_copy(tb.at[pl.ds(0, C2)], rows_b.at[b],
                              sems[b]).wait()

    lane = jnp.arange(16, dtype=jnp.int32)
    himask = jnp.full((16,), 0xFFFF0000, dtype=jnp.uint32)

    def compute(g, b):
        r2 = rows_b.at[b]

        def group_body(hi, res):
            for ei in range(8):
                e = hi * 8 + ei
                acc0 = jnp.zeros((16,), jnp.float32)
                acc1 = jnp.zeros((16,), jnp.float32)
                for k in range(DW // 16):
                    uu = plsc.bitcast(r2[e, pl.ds(k * 16, 16)], jnp.bfloat16)
                    tt = plsc.bitcast(r2[C + e, pl.ds(k * 16, 16)],
                                      jnp.bfloat16)
                    pw = plsc.bitcast(uu * tt, jnp.uint32)
                    a = plsc.bitcast(pw << 16, jnp.float32)
                    bb = plsc.bitcast(pw & himask, jnp.float32)
                    acc0 = acc0 + a
                    acc1 = acc1 + bb
                s = jnp.sum(acc0 + acc1)
                res = jnp.where(lane == (hi % 2) * 8 + ei, s, res)

            @pl.when(hi % 2 == 1)
            def _():
                out_v[g, pl.ds((hi // 2) * 16, 16)] = res

            return res

        lax.fori_loop(0, C // 8, group_body, jnp.zeros((16,), jnp.float32))

    for b in range(NBUF):
        fire(b, b)

    def outer(i, carry):
        g0 = i * NBUF
        for b in range(NBUF):
            g = g0 + b
            wait(b)
            compute(g, b)

            @pl.when(g + NBUF < NCHUNK)
            def _():
                fire(g + NBUF, b)
        return carry

    lax.fori_loop(0, NCHUNK // NBUF, outer, 0)

    pltpu.sync_copy(out_v, out_hbm.at[wid])


def _pack_table(h):
    hb = h.astype(jnp.bfloat16)
    return jax.lax.bitcast_convert_type(hb.reshape(h.shape[0], DW, 2),
                                        jnp.float32)


def kernel(h_user, h_track, edge_index):
    src = edge_index[0].astype(jnp.int32)
    dst = edge_index[1].astype(jnp.int32) + V
    pad = EP - E
    src = jnp.concatenate([src, jnp.zeros((pad,), jnp.int32)])
    dst = jnp.concatenate([dst, jnp.full((pad,), V, jnp.int32)])
    idx = jnp.concatenate([src.reshape(NW, NCHUNK, C),
                           dst.reshape(NW, NCHUNK, C)], axis=-1)
    tb = jnp.concatenate([_pack_table(h_user), _pack_table(h_track)])
    out = _dot_edges(tb, idx)
    return out.reshape(EP)[:E]


# speedup vs baseline: 1.0770x; 1.0770x over previous
import functools

import jax
import jax.numpy as jnp
from jax import lax
from jax.experimental import pallas as pl
from jax.experimental.pallas import tpu as pltpu
from jax.experimental.pallas import tpu_sc as plsc

E = 160000
D = 256
DW = D // 2
NC = 2
NS = 16
NW = NC * NS
EP = 163840
EW = EP // NW
C = 64
NCHUNK = EW // C
NBUF = 4

_mesh = plsc.VectorSubcoreMesh(core_axis_name="c", subcore_axis_name="s")


@functools.partial(
    pl.kernel,
    mesh=_mesh,
    compiler_params=pltpu.CompilerParams(use_tc_tiling_on_sc=False,
                                         needs_layout_passes=False),
    out_type=jax.ShapeDtypeStruct((NW, NCHUNK, C), jnp.float32),
    scratch_types=[
        pltpu.VMEM((NCHUNK, C), jnp.int32),
        pltpu.VMEM((NCHUNK, C), jnp.int32),
        pltpu.VMEM((NBUF, C, DW), jnp.float32),
        pltpu.VMEM((NBUF, C, DW), jnp.float32),
        pltpu.VMEM((NCHUNK, C), jnp.float32),
        pltpu.SemaphoreType.DMA,
        pltpu.SemaphoreType.DMA,
        pltpu.SemaphoreType.DMA,
        pltpu.SemaphoreType.DMA,
    ],
)
def _dot_edges(hu, ht, src_hbm, dst_hbm, out_hbm,
               src_v, dst_v, u_b, t_b, out_v, sem0, sem1, sem2, sem3):
    wid = lax.axis_index("s") * NC + lax.axis_index("c")
    sems = (sem0, sem1, sem2, sem3)

    pltpu.sync_copy(src_hbm.at[wid], src_v)
    pltpu.sync_copy(dst_hbm.at[wid], dst_v)

    def fire(g, b):
        pltpu.async_copy(hu.at[src_v.at[g]], u_b.at[b], sems[b])
        pltpu.async_copy(ht.at[dst_v.at[g]], t_b.at[b], sems[b])

    def wait(b):
        pltpu.make_async_copy(hu.at[pl.ds(0, C)], u_b.at[b], sems[b]).wait()
        pltpu.make_async_copy(ht.at[pl.ds(0, C)], t_b.at[b], sems[b]).wait()

    lane = jnp.arange(16, dtype=jnp.int32)
    himask = jnp.full((16,), 0xFFFF0000, dtype=jnp.uint32)

    def compute(g, b):
        u2 = u_b.at[b]
        t2 = t_b.at[b]

        def group_body(hi, res):
            for ei in range(8):
                e = hi * 8 + ei
                acc0 = jnp.zeros((16,), jnp.float32)
                acc1 = jnp.zeros((16,), jnp.float32)
                for k in range(DW // 16):
                    uu = plsc.bitcast(u2[e, pl.ds(k * 16, 16)], jnp.bfloat16)
                    tt = plsc.bitcast(t2[e, pl.ds(k * 16, 16)], jnp.bfloat16)
                    pw = plsc.bitcast(uu * tt, jnp.uint32)
                    a = plsc.bitcast(pw << 16, jnp.float32)
                    bb = plsc.bitcast(pw & himask, jnp.float32)
                    acc0 = acc0 + a
                    acc1 = acc1 + bb
                s = jnp.sum(acc0 + acc1)
                res = jnp.where(lane == (hi % 2) * 8 + ei, s, res)

            @pl.when(hi % 2 == 1)
            def _():
                out_v[g, pl.ds((hi // 2) * 16, 16)] = res

            return res

        lax.fori_loop(0, C // 8, group_body, jnp.zeros((16,), jnp.float32))

    for b in range(NBUF):
        fire(b, b)

    def outer(i, carry):
        g0 = i * NBUF
        for b in range(NBUF):
            g = g0 + b
            wait(b)
            compute(g, b)

            @pl.when(g + NBUF < NCHUNK)
            def _():
                fire(g + NBUF, b)
        return carry

    lax.fori_loop(0, NCHUNK // NBUF, outer, 0)

    pltpu.sync_copy(out_v, out_hbm.at[wid])


def _pack_table(h):
    hb = h.astype(jnp.bfloat16)
    return jax.lax.bitcast_convert_type(hb.reshape(h.shape[0], DW, 2),
                                        jnp.float32)


def kernel(h_user, h_track, edge_index):
    src = edge_index[0].astype(jnp.int32)
    dst = edge_index[1].astype(jnp.int32)
    pad = EP - E
    src = jnp.concatenate([src, jnp.zeros((pad,), jnp.int32)])
    dst = jnp.concatenate([dst, jnp.zeros((pad,), jnp.int32)])
    out = _dot_edges(_pack_table(h_user), _pack_table(h_track),
                     src.reshape(NW, NCHUNK, C), dst.reshape(NW, NCHUNK, C))
    return out.reshape(EP)[:E]


# speedup vs baseline: 1.4049x; 1.3044x over previous
import functools

import jax
import jax.numpy as jnp
from jax import lax
from jax.experimental import pallas as pl
from jax.experimental.pallas import tpu as pltpu
from jax.experimental.pallas import tpu_sc as plsc

E = 160000
D = 256
NC = 2
NS = 16
NW = NC * NS
EP = 163840
EW = EP // NW
C = 64
NCHUNK = EW // C
NBUF = 2

_mesh = plsc.VectorSubcoreMesh(core_axis_name="c", subcore_axis_name="s")


@functools.partial(
    pl.kernel,
    mesh=_mesh,
    compiler_params=pltpu.CompilerParams(use_tc_tiling_on_sc=False,
                                         needs_layout_passes=False),
    out_type=jax.ShapeDtypeStruct((NW, NCHUNK, C), jnp.float32),
    scratch_types=[
        pltpu.VMEM((NCHUNK, C), jnp.int32),
        pltpu.VMEM((NCHUNK, C), jnp.int32),
        pltpu.VMEM((NBUF, C, D), jnp.float32),
        pltpu.VMEM((NBUF, C, D), jnp.float32),
        pltpu.VMEM((NCHUNK, C), jnp.float32),
        pltpu.SemaphoreType.DMA,
        pltpu.SemaphoreType.DMA,
    ],
)
def _dot_edges(hu, ht, src_hbm, dst_hbm, out_hbm,
               src_v, dst_v, u_b, t_b, out_v, sem0, sem1):
    wid = lax.axis_index("s") * NC + lax.axis_index("c")
    sems = (sem0, sem1)

    pltpu.sync_copy(src_hbm.at[wid], src_v)
    pltpu.sync_copy(dst_hbm.at[wid], dst_v)

    def fire(g, b):
        pltpu.async_copy(hu.at[src_v.at[g]], u_b.at[b], sems[b])
        pltpu.async_copy(ht.at[dst_v.at[g]], t_b.at[b], sems[b])

    def wait(b):
        pltpu.make_async_copy(hu.at[pl.ds(0, C)], u_b.at[b], sems[b]).wait()
        pltpu.make_async_copy(ht.at[pl.ds(0, C)], t_b.at[b], sems[b]).wait()

    lane = jnp.arange(16, dtype=jnp.int32)

    def compute(g, b):
        u2 = u_b.at[b]
        t2 = t_b.at[b]

        def group_body(hi, res):
            for ei in range(8):
                e = hi * 8 + ei
                acc0 = jnp.zeros((16,), jnp.float32)
                acc1 = jnp.zeros((16,), jnp.float32)
                for k in range(0, D // 16, 2):
                    acc0 = acc0 + u2[e, pl.ds(k * 16, 16)] * t2[e, pl.ds(k * 16, 16)]
                    acc1 = acc1 + u2[e, pl.ds((k + 1) * 16, 16)] * t2[e, pl.ds((k + 1) * 16, 16)]
                s = jnp.sum(acc0 + acc1)
                res = jnp.where(lane == (hi % 2) * 8 + ei, s, res)

            @pl.when(hi % 2 == 1)
            def _():
                out_v[g, pl.ds((hi // 2) * 16, 16)] = res

            return res

        lax.fori_loop(0, C // 8, group_body, jnp.zeros((16,), jnp.float32))

    for b in range(NBUF):
        fire(b, b)

    def outer(i, carry):
        g0 = i * NBUF
        for b in range(NBUF):
            g = g0 + b
            wait(b)
            compute(g, b)

            @pl.when(g + NBUF < NCHUNK)
            def _():
                fire(g + NBUF, b)
        return carry

    lax.fori_loop(0, NCHUNK // NBUF, outer, 0)

    pltpu.sync_copy(out_v, out_hbm.at[wid])


def kernel(h_user, h_track, edge_index):
    src = edge_index[0].astype(jnp.int32)
    dst = edge_index[1].astype(jnp.int32)
    pad = EP - E
    src = jnp.concatenate([src, jnp.zeros((pad,), jnp.int32)])
    dst = jnp.concatenate([dst, jnp.zeros((pad,), jnp.int32)])
    out = _dot_edges(h_user, h_track,
                     src.reshape(NW, NCHUNK, C), dst.reshape(NW, NCHUNK, C))
    return out.reshape(EP)[:E]
